# baseline (speedup 1.0000x reference)
import functools
import jax
import jax.numpy as jnp
from jax import lax
from jax.experimental import pallas as pl
from jax.experimental.pallas import tpu as pltpu
from jax.experimental.pallas import tpu_sc as plsc

_B = 16384
_D = 32
_NC = 2
_NS = 16
_NW = _NC * _NS
_BPW = _B // _NW
_CH = 128
_NCH = _BPW // _CH

_MLP_TILE = 2048


def _gather_body(uid_hbm, bid_hbm, utab_hbm, btab_hbm, uout_hbm, bout_hbm,
                 uidx_v, bidx_v, urows_v, brows_v, usem, bsem):
    wid = lax.axis_index("s") * _NC + lax.axis_index("c")
    pltpu.sync_copy(uid_hbm.at[wid], uidx_v)
    pltpu.sync_copy(bid_hbm.at[wid], bidx_v)
    copies = []
    for j in range(_NCH):
        copies.append(pltpu.async_copy(utab_hbm.at[uidx_v.at[j]], urows_v.at[j], usem))
        copies.append(pltpu.async_copy(btab_hbm.at[bidx_v.at[j]], brows_v.at[j], bsem))
    for c in copies:
        c.wait()
    pltpu.sync_copy(urows_v, uout_hbm.at[wid])
    pltpu.sync_copy(brows_v, bout_hbm.at[wid])


@functools.lru_cache(maxsize=None)
def _sc_gather_fn():
    return pl.kernel(
        _gather_body,
        out_type=(
            jax.ShapeDtypeStruct((_NW, _NCH, _CH, _D), jnp.float32),
            jax.ShapeDtypeStruct((_NW, _NCH, _CH, _D), jnp.float32),
        ),
        mesh=plsc.VectorSubcoreMesh(core_axis_name="c", subcore_axis_name="s"),
        compiler_params=pltpu.CompilerParams(use_tc_tiling_on_sc=False),
        scratch_types=[
            pltpu.VMEM((_NCH, _CH), jnp.int32),
            pltpu.VMEM((_NCH, _CH), jnp.int32),
            pltpu.VMEM((_NCH, _CH, _D), jnp.float32),
            pltpu.VMEM((_NCH, _CH, _D), jnp.float32),
            pltpu.SemaphoreType.DMA,
            pltpu.SemaphoreType.DMA,
        ],
    )


def _mlp_body(u_ref, b_ref, w1u_ref, w1b_ref, b1_ref, w2_ref, b2_ref,
              w3_ref, b3_ref, w4_ref, b4_ref, out_ref):
    h = u_ref[...] @ w1u_ref[...] + b_ref[...] @ w1b_ref[...] + b1_ref[...]
    h = jnp.maximum(h, 0.0)
    h = jnp.maximum(h @ w2_ref[...] + b2_ref[...], 0.0)
    h = jnp.maximum(h @ w3_ref[...] + b3_ref[...], 0.0)
    out_ref[...] = h @ w4_ref[...] + b4_ref[...]


def _mlp(u_emb, b_emb, W1u, W1b, b1, W2, b2, W3, b3, W4, b4):
    n_tiles = _B // _MLP_TILE
    full = lambda shape: pl.BlockSpec(shape, lambda i: (0, 0))
    return pl.pallas_call(
        _mlp_body,
        grid=(n_tiles,),
        in_specs=[
            pl.BlockSpec((_MLP_TILE, _D), lambda i: (i, 0)),
            pl.BlockSpec((_MLP_TILE, _D), lambda i: (i, 0)),
            full(W1u.shape), full(W1b.shape), full(b1.shape),
            full(W2.shape), full(b2.shape),
            full(W3.shape), full(b3.shape),
            full(W4.shape), full(b4.shape),
        ],
        out_specs=pl.BlockSpec((_MLP_TILE, 1), lambda i: (i, 0)),
        out_shape=jax.ShapeDtypeStruct((_B, 1), jnp.float32),
    )(u_emb, b_emb, W1u, W1b, b1, W2, b2, W3, b3, W4, b4)


def kernel(user_id, book_title, user_table, book_table,
           W1, b1, W2, b2, W3, b3, W4, b4):
    uid3 = user_id.reshape(_NW, _NCH, _CH)
    bid3 = book_title.reshape(_NW, _NCH, _CH)
    u_emb, b_emb = _sc_gather_fn()(uid3, bid3, user_table, book_table)
    u_emb = u_emb.reshape(_B, _D)
    b_emb = b_emb.reshape(_B, _D)
    return _mlp(u_emb, b_emb,
                W1[:_D], W1[_D:], b1.reshape(1, -1),
                W2, b2.reshape(1, -1),
                W3, b3.reshape(1, -1),
                W4, b4.reshape(1, -1))
